# baseline (speedup 1.0000x reference)
import functools

import jax
import jax.numpy as jnp
from jax import lax
from jax.experimental import pallas as pl
from jax.experimental.pallas import tpu as pltpu
from jax.experimental.pallas import tpu_sc as plsc

N_EXP = 256
TOPK = 6
SCALE = 1.5
TBLK = 512
NW = 32
L = 16
TSTAGE = 128
GROUP_TILES = (3, 3, 2)
UNROLL = 4


def _score_block(x_ref, w_ref, b_ref, out_ref):
    scores = lax.dot_general(
        w_ref[...], x_ref[...],
        (((1,), (1,)), ((), ())),
        preferred_element_type=jnp.float32,
    )
    out_ref[...] = jnp.sqrt(jax.nn.softplus(scores)) + b_ref[...].reshape(N_EXP, 1)


def _insert(vs, ids, newv, newi):
    vs = list(vs)
    ids = list(ids)
    for j in range(TOPK):
        gt = newv > vs[j]
        vj = jnp.where(gt, newv, vs[j])
        ij = jnp.where(gt, newi, ids[j])
        newv = jnp.where(gt, vs[j], newv)
        newi = jnp.where(gt, ids[j], newi)
        vs[j] = vj
        ids[j] = ij
    return tuple(vs), tuple(ids)


def _make_topk_sc(tpw):
    def _topk_sc(bsc_hbm, bias_hbm, w_hbm, i_hbm, bs_v, bias_v, wout_v, iout_v):
        wid = lax.axis_index("s") * 2 + lax.axis_index("c")
        base = wid * tpw
        pltpu.sync_copy(bias_hbm, bias_v)

        lane = lax.broadcasted_iota(jnp.int32, (L,), 0)
        neg = jnp.full((L,), -jnp.inf, jnp.float32)
        zero = jnp.zeros((L,), jnp.int32)

        for sub in range(tpw // TSTAGE):
            pltpu.sync_copy(
                bsc_hbm.at[:, pl.ds(base + sub * TSTAGE, TSTAGE)], bs_v)

            g0 = 0
            for tile in GROUP_TILES:
                gs = tuple(range(g0, g0 + tile))
                g0 += tile

                def body(e, carry, gs=gs):
                    ei = jnp.full((L,), e, jnp.int32)
                    out = []
                    for k, g in enumerate(gs):
                        sv = bs_v[e, pl.ds(g * L, L)]
                        vk, ik = _insert(carry[2 * k], carry[2 * k + 1], sv, ei)
                        out.append(vk)
                        out.append(ik)
                    return tuple(out)

                init = ((neg,) * TOPK, (zero,) * TOPK) * tile
                fin = lax.fori_loop(0, N_EXP, body, init, unroll=UNROLL)

                for k, g in enumerate(gs):
                    vs, ids = fin[2 * k], fin[2 * k + 1]
                    ws = []
                    for j in range(TOPK):
                        bj = plsc.load_gather(bias_v, [ids[j]])
                        ws.append(vs[j] - bj)
                    tot = ws[0] + ws[1] + ws[2] + ws[3] + ws[4] + ws[5]
                    inv = SCALE / tot
                    rows = sub * TSTAGE + g * L + lane
                    for j in range(TOPK):
                        col = jnp.full((L,), j, jnp.int32)
                        plsc.store_scatter(wout_v, [rows, col], ws[j] * inv)
                        plsc.store_scatter(iout_v, [rows, col], ids[j])

        pltpu.sync_copy(wout_v, w_hbm.at[pl.ds(base, tpw)])
        pltpu.sync_copy(iout_v, i_hbm.at[pl.ds(base, tpw)])

    return _topk_sc


@jax.jit
def kernel(x, W, bias):
    n_tokens = x.shape[0]
    tpw = n_tokens // NW

    bsc = pl.pallas_call(
        _score_block,
        grid=(n_tokens // TBLK,),
        in_specs=[
            pl.BlockSpec((TBLK, x.shape[1]), lambda i: (i, 0)),
            pl.BlockSpec((N_EXP, x.shape[1]), lambda i: (0, 0)),
            pl.BlockSpec((N_EXP,), lambda i: (0,)),
        ],
        out_specs=pl.BlockSpec((N_EXP, TBLK), lambda i: (0, i)),
        out_shape=jax.ShapeDtypeStruct((N_EXP, n_tokens), jnp.float32),
    )(x, W, bias)

    mesh = plsc.VectorSubcoreMesh(core_axis_name="c", subcore_axis_name="s")
    topk = functools.partial(
        pl.kernel,
        mesh=mesh,
        out_type=[
            jax.ShapeDtypeStruct((n_tokens, TOPK), jnp.float32),
            jax.ShapeDtypeStruct((n_tokens, TOPK), jnp.int32),
        ],
        scratch_types=[
            pltpu.VMEM((N_EXP, TSTAGE), jnp.float32),
            pltpu.VMEM((N_EXP,), jnp.float32),
            pltpu.VMEM((tpw, TOPK), jnp.float32),
            pltpu.VMEM((tpw, TOPK), jnp.int32),
        ],
        compiler_params=pltpu.CompilerParams(
            needs_layout_passes=False, use_tc_tiling_on_sc=False),
    )(_make_topk_sc(tpw))
    wout, iout = topk(bsc, bias)
    return (wout, iout)


# speedup vs baseline: 1.0739x; 1.0739x over previous
import functools

import jax
import jax.numpy as jnp
from jax import lax
from jax.experimental import pallas as pl
from jax.experimental.pallas import tpu as pltpu
from jax.experimental.pallas import tpu_sc as plsc

N_EXP = 256
TOPK = 6
SCALE = 1.5
TBLK = 512
NW = 32
L = 16
TSTAGE = 128
GROUP_TILES = (3, 3, 2)
UNROLL = 4


def _score_block(x_ref, w_ref, b_ref, out_ref):
    scores = lax.dot_general(
        w_ref[...], x_ref[...],
        (((1,), (1,)), ((), ())),
        preferred_element_type=jnp.float32,
    )
    out_ref[...] = jnp.sqrt(jax.nn.softplus(scores)) + b_ref[...].reshape(N_EXP, 1)


def _insert(vs, ids, newv, newi):
    vs = list(vs)
    ids = list(ids)
    for j in range(TOPK):
        gt = newv > vs[j]
        vj = jnp.where(gt, newv, vs[j])
        ij = jnp.where(gt, newi, ids[j])
        newv = jnp.where(gt, vs[j], newv)
        newi = jnp.where(gt, ids[j], newi)
        vs[j] = vj
        ids[j] = ij
    return tuple(vs), tuple(ids)


def _make_topk_sc(tpw):
    def _topk_sc(bsc_hbm, bias_hbm, w_hbm, i_hbm, bs_v, bias_v, wout_v, iout_v):
        wid = lax.axis_index("s") * 2 + lax.axis_index("c")
        base = wid * tpw
        pltpu.sync_copy(bias_hbm, bias_v)

        lane = lax.broadcasted_iota(jnp.int32, (L,), 0)
        neg = jnp.full((L,), -jnp.inf, jnp.float32)
        zero = jnp.zeros((L,), jnp.int32)

        for sub in range(tpw // TSTAGE):
            pltpu.sync_copy(
                bsc_hbm.at[:, pl.ds(base + sub * TSTAGE, TSTAGE)], bs_v)

            g0 = 0
            for tile in GROUP_TILES:
                gs = tuple(range(g0, g0 + tile))
                g0 += tile

                def body(e, carry, gs=gs):
                    ei = jnp.full((L,), e, jnp.int32)
                    out = []
                    for k, g in enumerate(gs):
                        sv = bs_v[e, pl.ds(g * L, L)]
                        vk, ik = _insert(carry[2 * k], carry[2 * k + 1], sv, ei)
                        out.append(vk)
                        out.append(ik)
                    return tuple(out)

                init = ((neg,) * TOPK, (zero,) * TOPK) * tile
                fin = lax.fori_loop(0, N_EXP, body, init, unroll=UNROLL)

                for k, g in enumerate(gs):
                    vs, ids = fin[2 * k], fin[2 * k + 1]
                    ws = []
                    for j in range(TOPK):
                        bj = plsc.load_gather(bias_v, [ids[j]])
                        ws.append(vs[j] - bj)
                    tot = ws[0] + ws[1] + ws[2] + ws[3] + ws[4] + ws[5]
                    inv = SCALE / tot
                    rows = (sub * TSTAGE + g * L + lane) * TOPK
                    for j in range(TOPK):
                        plsc.store_scatter(wout_v, [rows + j], ws[j] * inv)
                        plsc.store_scatter(iout_v, [rows + j], ids[j])

        pltpu.sync_copy(wout_v, w_hbm.at[pl.ds(base * TOPK, tpw * TOPK)])
        pltpu.sync_copy(iout_v, i_hbm.at[pl.ds(base * TOPK, tpw * TOPK)])

    return _topk_sc


@jax.jit
def kernel(x, W, bias):
    n_tokens = x.shape[0]
    tpw = n_tokens // NW

    bsc = pl.pallas_call(
        _score_block,
        grid=(n_tokens // TBLK,),
        in_specs=[
            pl.BlockSpec((TBLK, x.shape[1]), lambda i: (i, 0)),
            pl.BlockSpec((N_EXP, x.shape[1]), lambda i: (0, 0)),
            pl.BlockSpec((N_EXP,), lambda i: (0,)),
        ],
        out_specs=pl.BlockSpec((N_EXP, TBLK), lambda i: (0, i)),
        out_shape=jax.ShapeDtypeStruct((N_EXP, n_tokens), jnp.float32),
    )(x, W, bias)

    mesh = plsc.VectorSubcoreMesh(core_axis_name="c", subcore_axis_name="s")
    topk = functools.partial(
        pl.kernel,
        mesh=mesh,
        out_type=[
            jax.ShapeDtypeStruct((n_tokens * TOPK,), jnp.float32),
            jax.ShapeDtypeStruct((n_tokens * TOPK,), jnp.int32),
        ],
        scratch_types=[
            pltpu.VMEM((N_EXP, TSTAGE), jnp.float32),
            pltpu.VMEM((N_EXP,), jnp.float32),
            pltpu.VMEM((tpw * TOPK,), jnp.float32),
            pltpu.VMEM((tpw * TOPK,), jnp.int32),
        ],
        compiler_params=pltpu.CompilerParams(
            needs_layout_passes=False, use_tc_tiling_on_sc=True),
    )(_make_topk_sc(tpw))
    wout, iout = topk(bsc, bias)
    return (wout.reshape(n_tokens, TOPK), iout.reshape(n_tokens, TOPK))


# speedup vs baseline: 1.2145x; 1.1310x over previous
import functools

import jax
import jax.numpy as jnp
from jax import lax
from jax.experimental import pallas as pl
from jax.experimental.pallas import tpu as pltpu
from jax.experimental.pallas import tpu_sc as plsc

N_EXP = 256
TOPK = 6
SCALE = 1.5
TBLK = 512
NW = 32
L = 16
TSTAGE = 128
NGRP = 4
UNROLL = 4


def _score_block(x_ref, w_ref, b_ref, out_ref):
    scores = lax.dot_general(
        w_ref[...], x_ref[...],
        (((1,), (1,)), ((), ())),
        preferred_element_type=jnp.float32,
    )
    out_ref[...] = jnp.sqrt(jax.nn.softplus(scores)) + b_ref[...].reshape(N_EXP, 1)


def _insert(vs, ids, newv, newi):
    vs = list(vs)
    ids = list(ids)
    for j in range(TOPK):
        gt = newv > vs[j]
        vj = jnp.where(gt, newv, vs[j])
        ij = jnp.where(gt, newi, ids[j])
        newv = jnp.where(gt, vs[j], newv)
        newi = jnp.where(gt, ids[j], newi)
        vs[j] = vj
        ids[j] = ij
    return tuple(vs), tuple(ids)


def _make_topk_sc(tpw):
    def _topk_sc(bsc_hbm, bias_hbm, w_hbm, i_hbm, bs_v, bias_v, wout_v, iout_v):
        wid = lax.axis_index("s") * 2 + lax.axis_index("c")
        base = wid * tpw
        pltpu.sync_copy(bias_hbm, bias_v)

        lane = lax.broadcasted_iota(jnp.int32, (L,), 0)
        neg = jnp.full((L,), -jnp.inf, jnp.float32)
        zero = jnp.zeros((L,), jnp.int32)

        for sub in range(tpw // TSTAGE):
            pltpu.sync_copy(
                bsc_hbm.at[:, pl.ds(base + sub * TSTAGE, TSTAGE)], bs_v)

            def tile_body(t, _unused, sub=sub):
                gbase = t * (NGRP * L)

                def body(e, carry):
                    ei = jnp.full((L,), e, jnp.int32)
                    out = []
                    for k in range(NGRP):
                        sv = bs_v[e, pl.ds(gbase + k * L, L)]
                        vk, ik = _insert(carry[2 * k], carry[2 * k + 1],
                                         sv, ei)
                        out.append(vk)
                        out.append(ik)
                    return tuple(out)

                init = ((neg,) * TOPK, (zero,) * TOPK) * NGRP
                fin = lax.fori_loop(0, N_EXP, body, init, unroll=UNROLL)

                for k in range(NGRP):
                    vs, ids = fin[2 * k], fin[2 * k + 1]
                    ws = []
                    for j in range(TOPK):
                        bj = plsc.load_gather(bias_v, [ids[j]])
                        ws.append(vs[j] - bj)
                    tot = ws[0] + ws[1] + ws[2] + ws[3] + ws[4] + ws[5]
                    inv = SCALE / tot
                    cols = sub * TSTAGE + gbase + k * L + lane
                    for j in range(TOPK):
                        row = jnp.full((L,), j, jnp.int32)
                        plsc.store_scatter(wout_v, [row, cols], ws[j] * inv)
                        plsc.store_scatter(iout_v, [row, cols], ids[j])
                return 0

            lax.fori_loop(0, TSTAGE // (NGRP * L), tile_body, 0)

        pltpu.sync_copy(wout_v, w_hbm.at[:, pl.ds(base, tpw)])
        pltpu.sync_copy(iout_v, i_hbm.at[:, pl.ds(base, tpw)])

    return _topk_sc


@jax.jit
def kernel(x, W, bias):
    n_tokens = x.shape[0]
    tpw = n_tokens // NW

    bsc = pl.pallas_call(
        _score_block,
        grid=(n_tokens // TBLK,),
        in_specs=[
            pl.BlockSpec((TBLK, x.shape[1]), lambda i: (i, 0)),
            pl.BlockSpec((N_EXP, x.shape[1]), lambda i: (0, 0)),
            pl.BlockSpec((N_EXP,), lambda i: (0,)),
        ],
        out_specs=pl.BlockSpec((N_EXP, TBLK), lambda i: (0, i)),
        out_shape=jax.ShapeDtypeStruct((N_EXP, n_tokens), jnp.float32),
    )(x, W, bias)

    mesh = plsc.VectorSubcoreMesh(core_axis_name="c", subcore_axis_name="s")
    topk = functools.partial(
        pl.kernel,
        mesh=mesh,
        out_type=[
            jax.ShapeDtypeStruct((TOPK, n_tokens), jnp.float32),
            jax.ShapeDtypeStruct((TOPK, n_tokens), jnp.int32),
        ],
        scratch_types=[
            pltpu.VMEM((N_EXP, TSTAGE), jnp.float32),
            pltpu.VMEM((N_EXP,), jnp.float32),
            pltpu.VMEM((TOPK, tpw), jnp.float32),
            pltpu.VMEM((TOPK, tpw), jnp.int32),
        ],
        compiler_params=pltpu.CompilerParams(
            needs_layout_passes=False, use_tc_tiling_on_sc=True),
    )(_make_topk_sc(tpw))
    wout_t, iout_t = topk(bsc, bias)
    return (wout_t.T, iout_t.T)
